# baseline (speedup 1.0000x reference)
import jax
import jax.numpy as jnp
from jax.experimental import pallas as pl
from jax.experimental.pallas import tpu as pltpu

_N = 50000
_C = 256
_G = 64
_EPS = 1e-05
_R = 2000
_NB = _N // _R


def _stats_kernel(x_ref, b_ref, o_ref, acc):
    i = pl.program_id(0)

    @pl.when(i == 0)
    def _():
        acc[...] = jnp.zeros_like(acc)

    xb = x_ref[...]
    ones_c = jnp.ones((_C, 1), jnp.float32)
    rs = jnp.dot(xb, ones_c, preferred_element_type=jnp.float32,
                 precision=jax.lax.Precision.HIGHEST)
    rq = jnp.dot(xb * xb, ones_c, preferred_element_type=jnp.float32,
                 precision=jax.lax.Precision.HIGHEST)
    b = b_ref[0, 0, :]
    seg = jax.lax.broadcasted_iota(jnp.int32, (_G, _R), 0)
    oh = (seg == b[None, :]).astype(jnp.float32)
    vals = jnp.concatenate([jnp.ones_like(rs), rs, rq], axis=1)
    acc[...] += jnp.dot(oh, vals, preferred_element_type=jnp.float32,
                        precision=jax.lax.Precision.HIGHEST)

    @pl.when(i == _NB - 1)
    def _():
        o_ref[...] = acc[...]


def _norm_kernel(x_ref, b_ref, s_ref, w_ref, bias_ref, o_ref):
    stats = s_ref[...]
    cnt = jnp.maximum(stats[:, 0:1], 1.0) * _C
    mean = stats[:, 1:2] / cnt
    var = jnp.maximum(stats[:, 2:3] / cnt - mean * mean, 0.0)
    inv = 1.0 / (jnp.sqrt(var) + _EPS)
    b = b_ref[0, 0, :]
    seg = jax.lax.broadcasted_iota(jnp.int32, (_R, _G), 1)
    oh = (seg == b[:, None]).astype(jnp.float32)
    mi = jnp.dot(oh, jnp.concatenate([mean, inv], axis=1),
                 preferred_element_type=jnp.float32,
                 precision=jax.lax.Precision.HIGHEST)
    xb = x_ref[...]
    o_ref[...] = ((xb - mi[:, 0:1]) * mi[:, 1:2]) * w_ref[...] + bias_ref[...]


def kernel(x, batch, weight, bias):
    batch3 = batch.astype(jnp.int32).reshape(_NB, 1, _R)

    stats = pl.pallas_call(
        _stats_kernel,
        grid=(_NB,),
        in_specs=[
            pl.BlockSpec((_R, _C), lambda i: (i, 0)),
            pl.BlockSpec((1, 1, _R), lambda i: (i, 0, 0)),
        ],
        out_specs=pl.BlockSpec((_G, 3), lambda i: (0, 0)),
        out_shape=jax.ShapeDtypeStruct((_G, 3), jnp.float32),
        scratch_shapes=[pltpu.VMEM((_G, 3), jnp.float32)],
    )(x, batch3)

    out = pl.pallas_call(
        _norm_kernel,
        grid=(_NB,),
        in_specs=[
            pl.BlockSpec((_R, _C), lambda i: (i, 0)),
            pl.BlockSpec((1, 1, _R), lambda i: (i, 0, 0)),
            pl.BlockSpec((_G, 3), lambda i: (0, 0)),
            pl.BlockSpec((1, _C), lambda i: (0, 0)),
            pl.BlockSpec((1, _C), lambda i: (0, 0)),
        ],
        out_specs=pl.BlockSpec((_R, _C), lambda i: (i, 0)),
        out_shape=jax.ShapeDtypeStruct((_N, _C), jnp.float32),
    )(x, batch3, stats, weight, bias)
    return out


# speedup vs baseline: 1.4797x; 1.4797x over previous
import jax
import jax.numpy as jnp
from jax.experimental import pallas as pl
from jax.experimental.pallas import tpu as pltpu

_N = 50000
_C = 256
_G = 64
_EPS = 1e-05
_R = 1000
_NB = _N // _R


def _stats_kernel(x_ref, b_ref, o_ref, acc, accn):
    i = pl.program_id(0)

    @pl.when(i == 0)
    def _():
        acc[...] = jnp.zeros_like(acc)
        accn[...] = jnp.zeros_like(accn)

    xb = x_ref[...]
    b = b_ref[0, 0, :]
    seg = jax.lax.broadcasted_iota(jnp.int32, (_G, _R), 0)
    oh = (seg == b[None, :]).astype(jnp.float32)
    vals = jnp.concatenate([xb, xb * xb], axis=1)
    acc[...] += jnp.dot(oh, vals, preferred_element_type=jnp.float32)
    accn[...] += jnp.dot(oh, jnp.ones((_R, 8), jnp.float32),
                         preferred_element_type=jnp.float32)

    @pl.when(i == _NB - 1)
    def _():
        s = jnp.sum(acc[:, :_C], axis=1, keepdims=True)
        q = jnp.sum(acc[:, _C:], axis=1, keepdims=True)
        o_ref[...] = jnp.concatenate([accn[:, 0:1], s, q], axis=1)


def _norm_kernel(x_ref, b_ref, s_ref, w_ref, bias_ref, o_ref):
    stats = s_ref[...]
    cnt = jnp.maximum(stats[:, 0:1], 1.0) * _C
    mean = stats[:, 1:2] / cnt
    var = jnp.maximum(stats[:, 2:3] / cnt - mean * mean, 0.0)
    inv = 1.0 / (jnp.sqrt(var) + _EPS)
    b = b_ref[0, 0, :]
    seg = jax.lax.broadcasted_iota(jnp.int32, (_R, _G), 1)
    oh = (seg == b[:, None]).astype(jnp.float32)
    mi = jnp.dot(oh, jnp.concatenate([mean, inv], axis=1),
                 preferred_element_type=jnp.float32)
    xb = x_ref[...]
    o_ref[...] = ((xb - mi[:, 0:1]) * mi[:, 1:2]) * w_ref[...] + bias_ref[...]


def kernel(x, batch, weight, bias):
    batch3 = batch.astype(jnp.int32).reshape(_NB, 1, _R)

    stats = pl.pallas_call(
        _stats_kernel,
        grid=(_NB,),
        in_specs=[
            pl.BlockSpec((_R, _C), lambda i: (i, 0)),
            pl.BlockSpec((1, 1, _R), lambda i: (i, 0, 0)),
        ],
        out_specs=pl.BlockSpec((_G, 3), lambda i: (0, 0)),
        out_shape=jax.ShapeDtypeStruct((_G, 3), jnp.float32),
        scratch_shapes=[pltpu.VMEM((_G, 2 * _C), jnp.float32),
                        pltpu.VMEM((_G, 8), jnp.float32)],
    )(x, batch3)

    out = pl.pallas_call(
        _norm_kernel,
        grid=(_NB,),
        in_specs=[
            pl.BlockSpec((_R, _C), lambda i: (i, 0)),
            pl.BlockSpec((1, 1, _R), lambda i: (i, 0, 0)),
            pl.BlockSpec((_G, 3), lambda i: (0, 0)),
            pl.BlockSpec((1, _C), lambda i: (0, 0)),
            pl.BlockSpec((1, _C), lambda i: (0, 0)),
        ],
        out_specs=pl.BlockSpec((_R, _C), lambda i: (i, 0)),
        out_shape=jax.ShapeDtypeStruct((_N, _C), jnp.float32),
    )(x, batch3, stats, weight, bias)
    return out


# speedup vs baseline: 1.8115x; 1.2242x over previous
import jax
import jax.numpy as jnp
from jax.experimental import pallas as pl
from jax.experimental.pallas import tpu as pltpu

_N = 50000
_C = 256
_G = 64
_EPS = 1e-05
_R = 1000
_NB = _N // _R


def _ln_kernel(x_ref, b_ref, w_ref, bias_ref, o_ref, xcache, acc, accn, mi_tbl):
    p = pl.program_id(0)
    i = pl.program_id(1)

    @pl.when(jnp.logical_and(p == 0, i == 0))
    def _():
        acc[...] = jnp.zeros_like(acc)
        accn[...] = jnp.zeros_like(accn)

    b = b_ref[0, 0, :]

    @pl.when(p == 0)
    def _():
        xb = x_ref[...]
        xcache[pl.ds(i * _R, _R), :] = xb
        seg = jax.lax.broadcasted_iota(jnp.int32, (_G, _R), 0)
        oh = (seg == b[None, :]).astype(jnp.float32)
        vals = jnp.concatenate([xb, xb * xb], axis=1)
        acc[...] += jnp.dot(oh, vals, preferred_element_type=jnp.float32)
        accn[...] += jnp.dot(oh, jnp.ones((_R, 8), jnp.float32),
                             preferred_element_type=jnp.float32)

        @pl.when(i == _NB - 1)
        def _():
            cnt = jnp.maximum(accn[:, 0:1], 1.0) * _C
            s = jnp.sum(acc[:, :_C], axis=1, keepdims=True)
            q = jnp.sum(acc[:, _C:], axis=1, keepdims=True)
            mean = s / cnt
            var = jnp.maximum(q / cnt - mean * mean, 0.0)
            inv = 1.0 / (jnp.sqrt(var) + _EPS)
            mi_tbl[...] = jnp.concatenate([mean, inv], axis=1)

    @pl.when(p == 1)
    def _():
        xb = xcache[pl.ds(i * _R, _R), :]
        seg = jax.lax.broadcasted_iota(jnp.int32, (_R, _G), 1)
        oh = (seg == b[:, None]).astype(jnp.float32)
        mi = jnp.dot(oh, mi_tbl[...],
                     preferred_element_type=jnp.float32)
        o_ref[...] = ((xb - mi[:, 0:1]) * mi[:, 1:2]) * w_ref[...] + bias_ref[...]


def kernel(x, batch, weight, bias):
    batch3 = batch.astype(jnp.int32).reshape(_NB, 1, _R)

    out = pl.pallas_call(
        _ln_kernel,
        grid=(2, _NB),
        in_specs=[
            pl.BlockSpec((_R, _C), lambda p, i: (i * (1 - p), 0)),
            pl.BlockSpec((1, 1, _R), lambda p, i: (i, 0, 0)),
            pl.BlockSpec((1, _C), lambda p, i: (0, 0)),
            pl.BlockSpec((1, _C), lambda p, i: (0, 0)),
        ],
        out_specs=pl.BlockSpec((_R, _C), lambda p, i: (i * p, 0)),
        out_shape=jax.ShapeDtypeStruct((_N, _C), jnp.float32),
        scratch_shapes=[
            pltpu.VMEM((_N, _C), jnp.float32),
            pltpu.VMEM((_G, 2 * _C), jnp.float32),
            pltpu.VMEM((_G, 8), jnp.float32),
            pltpu.VMEM((_G, 2), jnp.float32),
        ],
    )(x, batch3, weight, bias)
    return out
